# baseline (speedup 1.0000x reference)
import functools

import jax
import jax.numpy as jnp
from jax import lax
from jax.experimental import pallas as pl
from jax.experimental.pallas import tpu as pltpu
from jax.experimental.pallas import tpu_sc as plsc

N = 10000
D_IN = 128
D_HID = 15
D_OUT = 128

NC = 2
NS = 16
NW = NC * NS
LANES = 16

NPAD = 10240
ROWS_PT = NPAD // NS
DUMMY = N

GROUP = 128
G_INNER = 8
CHUNK = GROUP * G_INNER

_MESH = plsc.VectorSubcoreMesh(
    core_axis_name="c", subcore_axis_name="s", num_cores=NC, num_subcores=NS)
_SC_PARAMS = pltpu.CompilerParams(use_tc_tiling_on_sc=False)


def _pad_chunks(e_total: int) -> int:
    return -(-e_total // (CHUNK * NW)) * CHUNK * NW


def _sc_degree(dst2d: jax.Array) -> jax.Array:
    gpt = dst2d.shape[0] // NW
    cpt = gpt // G_INNER

    @functools.partial(
        pl.kernel,
        out_type=jax.ShapeDtypeStruct((NC * NPAD, LANES), jnp.float32),
        mesh=_MESH,
        compiler_params=_SC_PARAMS,
        scratch_types=dict(
            acc=pltpu.VMEM_SHARED((NPAD, LANES), jnp.float32),
            didx=pltpu.VMEM((gpt, GROUP), jnp.int32),
            ones=pltpu.VMEM((GROUP, LANES), jnp.float32),
            zbuf=pltpu.VMEM((ROWS_PT, LANES), jnp.float32),
            ssem=pltpu.SemaphoreType.DMA,
        ),
    )
    def kern(dst_hbm, out_hbm, acc, didx, ones, zbuf, ssem):
        cid = lax.axis_index("c")
        sid = lax.axis_index("s")
        wid = cid * NS + sid

        def fill(i, _):
            zbuf[i, :] = jnp.zeros((LANES,), jnp.float32)
            return 0
        lax.fori_loop(0, ROWS_PT, fill, 0)

        def fill1(i, _):
            ones[i, :] = jnp.full((LANES,), 1.0, jnp.float32)
            return 0
        lax.fori_loop(0, GROUP, fill1, 0)

        pltpu.sync_copy(zbuf, acc.at[pl.ds(sid * ROWS_PT, ROWS_PT)])
        pltpu.sync_copy(dst_hbm.at[pl.ds(wid * gpt, gpt)], didx)
        plsc.subcore_barrier()

        def step(t, _):
            scat = [
                pltpu.async_copy(ones, acc.at[didx.at[t * G_INNER + j]], ssem,
                                 add=True)
                for j in range(G_INNER)
            ]
            for c in scat:
                c.wait()
            return 0
        lax.fori_loop(0, cpt, step, 0)
        plsc.subcore_barrier()

        pltpu.sync_copy(acc.at[pl.ds(sid * ROWS_PT, ROWS_PT)], zbuf)
        pltpu.sync_copy(zbuf, out_hbm.at[pl.ds(cid * NPAD + sid * ROWS_PT, ROWS_PT)])

    return kern(dst2d)


def _sc_aggregate(src2d: jax.Array, dst2d: jax.Array, g: jax.Array) -> jax.Array:
    gpt = src2d.shape[0] // NW
    cpt = gpt // G_INNER

    @functools.partial(
        pl.kernel,
        out_type=jax.ShapeDtypeStruct((NC * NPAD, LANES), jnp.float32),
        mesh=_MESH,
        compiler_params=_SC_PARAMS,
        scratch_types=dict(
            acc=pltpu.VMEM_SHARED((NPAD, LANES), jnp.float32),
            sidx=pltpu.VMEM((gpt, GROUP), jnp.int32),
            didx=pltpu.VMEM((gpt, GROUP), jnp.int32),
            rows=pltpu.VMEM((G_INNER, GROUP, LANES), jnp.float32),
            zbuf=pltpu.VMEM((ROWS_PT, LANES), jnp.float32),
            gsem=pltpu.SemaphoreType.DMA,
            ssem=pltpu.SemaphoreType.DMA,
        ),
    )
    def kern(src_hbm, dst_hbm, g_hbm, out_hbm,
             acc, sidx, didx, rows, zbuf, gsem, ssem):
        cid = lax.axis_index("c")
        sid = lax.axis_index("s")
        wid = cid * NS + sid

        def fill(i, _):
            zbuf[i, :] = jnp.zeros((LANES,), jnp.float32)
            return 0
        lax.fori_loop(0, ROWS_PT, fill, 0)
        pltpu.sync_copy(zbuf, acc.at[pl.ds(sid * ROWS_PT, ROWS_PT)])
        pltpu.sync_copy(src_hbm.at[pl.ds(wid * gpt, gpt)], sidx)
        pltpu.sync_copy(dst_hbm.at[pl.ds(wid * gpt, gpt)], didx)
        plsc.subcore_barrier()

        def step(t, _):
            gath = [
                pltpu.async_copy(g_hbm.at[sidx.at[t * G_INNER + j]],
                                 rows.at[j], gsem)
                for j in range(G_INNER)
            ]
            for c in gath:
                c.wait()
            scat = [
                pltpu.async_copy(rows.at[j], acc.at[didx.at[t * G_INNER + j]],
                                 ssem, add=True)
                for j in range(G_INNER)
            ]
            for c in scat:
                c.wait()
            return 0
        lax.fori_loop(0, cpt, step, 0)
        plsc.subcore_barrier()

        pltpu.sync_copy(acc.at[pl.ds(sid * ROWS_PT, ROWS_PT)], zbuf)
        pltpu.sync_copy(zbuf, out_hbm.at[pl.ds(cid * NPAD + sid * ROWS_PT, ROWS_PT)])

    return kern(src2d, dst2d, g)


def _dis_from_parts(degp):
    deg = degp[:NPAD, :1] + degp[NPAD:, :1] + 1.0
    return lax.rsqrt(deg)


def _tc_scale_matmul(x, w1p, degp):
    def body(x_ref, w_ref, d_ref, o_ref):
        dis = _dis_from_parts(d_ref[...])
        xw = jnp.dot(x_ref[...], w_ref[...], preferred_element_type=jnp.float32)
        o_ref[:N, :] = dis[:N] * xw
        o_ref[N:, :] = jnp.zeros((NPAD - N, LANES), jnp.float32)
    return pl.pallas_call(
        body,
        out_shape=jax.ShapeDtypeStruct((NPAD, LANES), jnp.float32),
    )(x, w1p, degp)


def _tc_mid(aggp, g1, degp, b1p):
    def body(a_ref, g_ref, d_ref, b_ref, o_ref):
        dis = _dis_from_parts(d_ref[...])
        agg = a_ref[:NPAD, :] + a_ref[NPAD:, :] + g_ref[...]
        h = jnp.maximum(dis * agg + b_ref[...], 0.0)
        o_ref[...] = dis * h
    return pl.pallas_call(
        body,
        out_shape=jax.ShapeDtypeStruct((NPAD, LANES), jnp.float32),
    )(aggp, g1, degp, b1p)


def _tc_out(aggp, g2, degp, w2p, b2p):
    def body(a_ref, g_ref, d_ref, w_ref, b_ref, o_ref):
        dis = _dis_from_parts(d_ref[...])
        agg = dis[:N] * (a_ref[:N, :] + a_ref[NPAD:NPAD + N, :] + g_ref[:N, :])
        o_ref[...] = jnp.dot(agg, w_ref[...],
                             preferred_element_type=jnp.float32) + b_ref[...]
    return pl.pallas_call(
        body,
        out_shape=jax.ShapeDtypeStruct((N, D_OUT), jnp.float32),
    )(aggp, g2, degp, w2p, b2p)


def kernel(x, edge_index, W1, b1, W2, b2):
    e = edge_index.shape[1]
    epad = _pad_chunks(e)
    pad = epad - e

    src = jnp.concatenate(
        [edge_index[0].astype(jnp.int32), jnp.zeros((pad,), jnp.int32)])
    dst = jnp.concatenate(
        [edge_index[1].astype(jnp.int32),
         DUMMY + jnp.arange(pad, dtype=jnp.int32) % (NPAD - N)])
    src2d = src.reshape(-1, GROUP)
    dst2d = dst.reshape(-1, GROUP)

    w1p = jnp.zeros((D_IN, LANES), jnp.float32).at[:, :D_HID].set(W1)
    b1p = jnp.zeros((1, LANES), jnp.float32).at[0, :D_HID].set(b1)
    w2p = jnp.zeros((LANES, D_OUT), jnp.float32).at[:D_HID].set(W2)
    b2p = b2.reshape(1, D_OUT)

    degp = _sc_degree(dst2d)
    g1 = _tc_scale_matmul(x, w1p, degp)
    agg1 = _sc_aggregate(src2d, dst2d, g1)
    g2 = _tc_mid(agg1, g1, degp, b1p)
    agg2 = _sc_aggregate(src2d, dst2d, g2)
    return _tc_out(agg2, g2, degp, w2p, b2p)


# speedup vs baseline: 1.5407x; 1.5407x over previous
import functools

import jax
import jax.numpy as jnp
from jax import lax
from jax.experimental import pallas as pl
from jax.experimental.pallas import tpu as pltpu
from jax.experimental.pallas import tpu_sc as plsc

N = 10000
D_IN = 128
D_HID = 15
D_OUT = 128

NC = 2
NS = 16
NW = NC * NS
LANES = 16

NPAD = 10240
ROWS_PT = NPAD // NS
DUMMY = N

GROUP = 128
G_INNER = 8
CHUNK = GROUP * G_INNER

_MESH = plsc.VectorSubcoreMesh(
    core_axis_name="c", subcore_axis_name="s", num_cores=NC, num_subcores=NS)
_SC_PARAMS = pltpu.CompilerParams(use_tc_tiling_on_sc=False)


def _pad_chunks(e_total: int) -> int:
    return -(-e_total // (CHUNK * NW)) * CHUNK * NW


def _sc_degree(dst2d: jax.Array) -> jax.Array:
    gpt = dst2d.shape[0] // NW
    cpt = gpt // G_INNER

    @functools.partial(
        pl.kernel,
        out_type=jax.ShapeDtypeStruct((NC * NPAD,), jnp.float32),
        mesh=_MESH,
        compiler_params=_SC_PARAMS,
        scratch_types=dict(
            acc=pltpu.VMEM_SHARED((NPAD,), jnp.float32),
            didx=pltpu.VMEM((gpt, GROUP), jnp.int32),
            ones=pltpu.VMEM((GROUP,), jnp.float32),
            zbuf=pltpu.VMEM((ROWS_PT,), jnp.float32),
            ssem=pltpu.SemaphoreType.DMA,
        ),
    )
    def kern(dst_hbm, out_hbm, acc, didx, ones, zbuf, ssem):
        cid = lax.axis_index("c")
        sid = lax.axis_index("s")
        wid = cid * NS + sid

        def fill(i, _):
            zbuf[pl.ds(i * LANES, LANES)] = jnp.zeros((LANES,), jnp.float32)
            return 0
        lax.fori_loop(0, ROWS_PT // LANES, fill, 0)

        def fill1(i, _):
            ones[pl.ds(i * LANES, LANES)] = jnp.full((LANES,), 1.0, jnp.float32)
            return 0
        lax.fori_loop(0, GROUP // LANES, fill1, 0)

        pltpu.sync_copy(zbuf, acc.at[pl.ds(sid * ROWS_PT, ROWS_PT)])
        pltpu.sync_copy(dst_hbm.at[pl.ds(wid * gpt, gpt)], didx)
        plsc.subcore_barrier()

        def step(t, _):
            scat = [
                pltpu.async_copy(ones, acc.at[didx.at[t * G_INNER + j]], ssem,
                                 add=True)
                for j in range(G_INNER)
            ]
            for c in scat:
                c.wait()
            return 0
        lax.fori_loop(0, cpt, step, 0)
        plsc.subcore_barrier()

        pltpu.sync_copy(acc.at[pl.ds(sid * ROWS_PT, ROWS_PT)], zbuf)
        pltpu.sync_copy(zbuf, out_hbm.at[pl.ds(cid * NPAD + sid * ROWS_PT, ROWS_PT)])

    return kern(dst2d)


def _sc_aggregate(src2d: jax.Array, dst2d: jax.Array, g: jax.Array) -> jax.Array:
    gpt = src2d.shape[0] // NW
    cpt = gpt // G_INNER

    @functools.partial(
        pl.kernel,
        out_type=jax.ShapeDtypeStruct((NC * NPAD, LANES), jnp.float32),
        mesh=_MESH,
        compiler_params=_SC_PARAMS,
        scratch_types=dict(
            acc=pltpu.VMEM_SHARED((NPAD, LANES), jnp.float32),
            gsh=pltpu.VMEM_SHARED((NPAD, LANES), jnp.float32),
            sidx=pltpu.VMEM((gpt, GROUP), jnp.int32),
            didx=pltpu.VMEM((gpt, GROUP), jnp.int32),
            rows=pltpu.VMEM((G_INNER, GROUP, LANES), jnp.float32),
            zbuf=pltpu.VMEM((ROWS_PT, LANES), jnp.float32),
            gsem=pltpu.SemaphoreType.DMA,
            ssem=pltpu.SemaphoreType.DMA,
        ),
    )
    def kern(src_hbm, dst_hbm, g_hbm, out_hbm,
             acc, gsh, sidx, didx, rows, zbuf, gsem, ssem):
        cid = lax.axis_index("c")
        sid = lax.axis_index("s")
        wid = cid * NS + sid

        def fill(i, _):
            zbuf[i, :] = jnp.zeros((LANES,), jnp.float32)
            return 0
        lax.fori_loop(0, ROWS_PT, fill, 0)
        pltpu.sync_copy(zbuf, acc.at[pl.ds(sid * ROWS_PT, ROWS_PT)])
        pltpu.sync_copy(g_hbm.at[pl.ds(sid * ROWS_PT, ROWS_PT)],
                        gsh.at[pl.ds(sid * ROWS_PT, ROWS_PT)])
        pltpu.sync_copy(src_hbm.at[pl.ds(wid * gpt, gpt)], sidx)
        pltpu.sync_copy(dst_hbm.at[pl.ds(wid * gpt, gpt)], didx)
        plsc.subcore_barrier()

        def step(t, _):
            gath = [
                pltpu.async_copy(gsh.at[sidx.at[t * G_INNER + j]],
                                 rows.at[j], gsem)
                for j in range(G_INNER)
            ]
            for c in gath:
                c.wait()
            scat = [
                pltpu.async_copy(rows.at[j], acc.at[didx.at[t * G_INNER + j]],
                                 ssem, add=True)
                for j in range(G_INNER)
            ]
            for c in scat:
                c.wait()
            return 0
        lax.fori_loop(0, cpt, step, 0)
        plsc.subcore_barrier()

        pltpu.sync_copy(acc.at[pl.ds(sid * ROWS_PT, ROWS_PT)], zbuf)
        pltpu.sync_copy(zbuf, out_hbm.at[pl.ds(cid * NPAD + sid * ROWS_PT, ROWS_PT)])

    return kern(src2d, dst2d, g)


def _dis_from_parts(degp):
    deg = degp[:NPAD].reshape(NPAD, 1) + degp[NPAD:].reshape(NPAD, 1) + 1.0
    return lax.rsqrt(deg)


def _tc_matmul1(x, w1p):
    def body(x_ref, w_ref, o_ref):
        o_ref[...] = jnp.dot(x_ref[...], w_ref[...],
                             preferred_element_type=jnp.float32)
    return pl.pallas_call(
        body,
        out_shape=jax.ShapeDtypeStruct((N, LANES), jnp.float32),
    )(x, w1p)


def _tc_scale(h1, degp):
    def body(h_ref, d_ref, o_ref):
        dis = _dis_from_parts(d_ref[...])
        o_ref[:N, :] = dis[:N] * h_ref[...]
        o_ref[N:, :] = jnp.zeros((NPAD - N, LANES), jnp.float32)
    return pl.pallas_call(
        body,
        out_shape=jax.ShapeDtypeStruct((NPAD, LANES), jnp.float32),
    )(h1, degp)


def _tc_mid(aggp, g1, degp, b1p):
    def body(a_ref, g_ref, d_ref, b_ref, o_ref):
        dis = _dis_from_parts(d_ref[...])
        agg = a_ref[:NPAD, :] + a_ref[NPAD:, :] + g_ref[...]
        h = jnp.maximum(dis * agg + b_ref[...], 0.0)
        o_ref[...] = dis * h
    return pl.pallas_call(
        body,
        out_shape=jax.ShapeDtypeStruct((NPAD, LANES), jnp.float32),
    )(aggp, g1, degp, b1p)


def _tc_out(aggp, g2, degp, w2p, b2p):
    def body(a_ref, g_ref, d_ref, w_ref, b_ref, o_ref):
        dis = _dis_from_parts(d_ref[...])
        agg = dis[:N] * (a_ref[:N, :] + a_ref[NPAD:NPAD + N, :] + g_ref[:N, :])
        o_ref[...] = jnp.dot(agg, w_ref[...],
                             preferred_element_type=jnp.float32) + b_ref[...]
    return pl.pallas_call(
        body,
        out_shape=jax.ShapeDtypeStruct((N, D_OUT), jnp.float32),
    )(aggp, g2, degp, w2p, b2p)


def kernel(x, edge_index, W1, b1, W2, b2):
    e = edge_index.shape[1]
    epad = _pad_chunks(e)
    pad = epad - e

    src = jnp.concatenate(
        [edge_index[0].astype(jnp.int32), jnp.zeros((pad,), jnp.int32)])
    dst = jnp.concatenate(
        [edge_index[1].astype(jnp.int32),
         DUMMY + jnp.arange(pad, dtype=jnp.int32) % (NPAD - N)])
    src2d = src.reshape(-1, GROUP)
    dst2d = dst.reshape(-1, GROUP)

    w1p = jnp.zeros((D_IN, LANES), jnp.float32).at[:, :D_HID].set(W1)
    b1p = jnp.zeros((1, LANES), jnp.float32).at[0, :D_HID].set(b1)
    w2p = jnp.zeros((LANES, D_OUT), jnp.float32).at[:D_HID].set(W2)
    b2p = b2.reshape(1, D_OUT)

    h1 = _tc_matmul1(x, w1p)
    degp = _sc_degree(dst2d)
    g1 = _tc_scale(h1, degp)
    agg1 = _sc_aggregate(src2d, dst2d, g1)
    g2 = _tc_mid(agg1, g1, degp, b1p)
    agg2 = _sc_aggregate(src2d, dst2d, g2)
    return _tc_out(agg2, g2, degp, w2p, b2p)


# speedup vs baseline: 1.6016x; 1.0395x over previous
import functools

import jax
import jax.numpy as jnp
from jax import lax
from jax.experimental import pallas as pl
from jax.experimental.pallas import tpu as pltpu
from jax.experimental.pallas import tpu_sc as plsc

N = 10000
D_IN = 128
D_HID = 15
D_OUT = 128

NC = 2
NS = 16
NW = NC * NS
LANES = 16

NPAD = 10240
ROWS_PT = NPAD // NS
DUMMY = N

GROUP = 128
G_INNER = 8
CHUNK = GROUP * G_INNER

_MESH = plsc.VectorSubcoreMesh(
    core_axis_name="c", subcore_axis_name="s", num_cores=NC, num_subcores=NS)
_SC_PARAMS = pltpu.CompilerParams(use_tc_tiling_on_sc=False,
                                  needs_layout_passes=False)


def _pad_chunks(e_total: int) -> int:
    return -(-e_total // (CHUNK * NW)) * CHUNK * NW


def _sc_degree(dst2d: jax.Array) -> jax.Array:
    gpt = dst2d.shape[0] // NW
    cpt = gpt // G_INNER

    @functools.partial(
        pl.kernel,
        out_type=jax.ShapeDtypeStruct((NC * NPAD,), jnp.float32),
        mesh=_MESH,
        compiler_params=_SC_PARAMS,
        scratch_types=dict(
            acc=pltpu.VMEM_SHARED((NPAD,), jnp.float32),
            didx=pltpu.VMEM((gpt, GROUP), jnp.int32),
            ones=pltpu.VMEM((GROUP,), jnp.float32),
            zbuf=pltpu.VMEM((ROWS_PT,), jnp.float32),
            ssem=pltpu.SemaphoreType.DMA,
        ),
    )
    def kern(dst_hbm, out_hbm, acc, didx, ones, zbuf, ssem):
        cid = lax.axis_index("c")
        sid = lax.axis_index("s")
        wid = cid * NS + sid

        def fill(i, _):
            zbuf[pl.ds(i * LANES, LANES)] = jnp.zeros((LANES,), jnp.float32)
            return 0
        lax.fori_loop(0, ROWS_PT // LANES, fill, 0)

        def fill1(i, _):
            ones[pl.ds(i * LANES, LANES)] = jnp.full((LANES,), 1.0, jnp.float32)
            return 0
        lax.fori_loop(0, GROUP // LANES, fill1, 0)

        pltpu.sync_copy(zbuf, acc.at[pl.ds(sid * ROWS_PT, ROWS_PT)])
        pltpu.sync_copy(dst_hbm.at[pl.ds(wid * gpt, gpt)], didx)
        plsc.subcore_barrier()

        def step(t, _):
            scat = [
                pltpu.async_copy(ones, acc.at[didx.at[t * G_INNER + j]], ssem,
                                 add=True)
                for j in range(G_INNER)
            ]
            for c in scat:
                c.wait()
            return 0
        lax.fori_loop(0, cpt, step, 0)
        plsc.subcore_barrier()

        pltpu.sync_copy(acc.at[pl.ds(sid * ROWS_PT, ROWS_PT)], zbuf)
        pltpu.sync_copy(zbuf, out_hbm.at[pl.ds(cid * NPAD + sid * ROWS_PT, ROWS_PT)])

    return kern(dst2d)


def _fisr(d):
    i = lax.bitcast_convert_type(d, jnp.int32)
    y = lax.bitcast_convert_type(jnp.int32(0x5F3759DF) - (i >> 1), jnp.float32)
    for _ in range(3):
        y = y * (1.5 - 0.5 * d * y * y)
    return y


def _sc_aggregate(src2d, dst2d, degp, h1, agg1p, b1v, layer: int):
    gpt = src2d.shape[0] // NW
    cpt = gpt // G_INNER

    scratch = dict(
        acc=pltpu.VMEM_SHARED((NPAD, LANES), jnp.float32),
        gsh=pltpu.VMEM_SHARED((NPAD, LANES), jnp.float32),
        sidx=pltpu.VMEM((gpt, GROUP), jnp.int32),
        didx=pltpu.VMEM((gpt, GROUP), jnp.int32),
        rows=pltpu.VMEM((G_INNER, GROUP, LANES), jnp.float32),
        zbuf=pltpu.VMEM((ROWS_PT, LANES), jnp.float32),
        hbuf=pltpu.VMEM((ROWS_PT, LANES), jnp.float32),
        d0v=pltpu.VMEM((ROWS_PT,), jnp.float32),
        d1v=pltpu.VMEM((ROWS_PT,), jnp.float32),
        disv=pltpu.VMEM((ROWS_PT,), jnp.float32),
        gsem=pltpu.SemaphoreType.DMA,
        ssem=pltpu.SemaphoreType.DMA,
    )
    if layer == 2:
        scratch.update(
            p0=pltpu.VMEM((ROWS_PT, LANES), jnp.float32),
            p1=pltpu.VMEM((ROWS_PT, LANES), jnp.float32),
            b1b=pltpu.VMEM((LANES,), jnp.float32),
        )

    @functools.partial(
        pl.kernel,
        out_type=jax.ShapeDtypeStruct((NC * NPAD, LANES), jnp.float32),
        mesh=_MESH,
        compiler_params=_SC_PARAMS,
        scratch_types=scratch,
    )
    def kern(*refs, acc, gsh, sidx, didx, rows, zbuf, hbuf, d0v, d1v, disv,
             gsem, ssem, p0=None, p1=None, b1b=None):
        if layer == 1:
            src_hbm, dst_hbm, degp_hbm, h1_hbm, out_hbm = refs
            a1_hbm = b1_hbm = None
        else:
            src_hbm, dst_hbm, degp_hbm, h1_hbm, a1_hbm, b1_hbm, out_hbm = refs
        cid = lax.axis_index("c")
        sid = lax.axis_index("s")
        wid = cid * NS + sid
        row0 = sid * ROWS_PT

        def fill(i, _):
            zbuf[i, :] = jnp.zeros((LANES,), jnp.float32)
            return 0
        lax.fori_loop(0, ROWS_PT, fill, 0)
        pltpu.sync_copy(zbuf, acc.at[pl.ds(row0, ROWS_PT)])

        pltpu.sync_copy(degp_hbm.at[pl.ds(row0, ROWS_PT)], d0v)
        pltpu.sync_copy(degp_hbm.at[pl.ds(NPAD + row0, ROWS_PT)], d1v)
        pltpu.sync_copy(h1_hbm.at[pl.ds(row0, ROWS_PT)], hbuf)
        if layer == 2:
            pltpu.sync_copy(a1_hbm.at[pl.ds(row0, ROWS_PT)], p0)
            pltpu.sync_copy(a1_hbm.at[pl.ds(NPAD + row0, ROWS_PT)], p1)
            pltpu.sync_copy(b1_hbm, b1b)

        def dfill(i, _):
            d = d0v[pl.ds(i * LANES, LANES)] + d1v[pl.ds(i * LANES, LANES)] + 1.0
            disv[pl.ds(i * LANES, LANES)] = _fisr(d)
            return 0
        lax.fori_loop(0, ROWS_PT // LANES, dfill, 0)

        def srow(r, _):
            db = plsc.load_gather(disv, [jnp.full((LANES,), r, jnp.int32)])
            g1r = db * hbuf[r, :]
            if layer == 1:
                zbuf[r, :] = g1r
            else:
                v = (p0[r, :] + p1[r, :] + g1r) * db + b1b[:]
                zbuf[r, :] = jnp.maximum(v, 0.0) * db
            return 0
        lax.fori_loop(0, ROWS_PT, srow, 0)
        pltpu.sync_copy(zbuf, gsh.at[pl.ds(row0, ROWS_PT)])

        pltpu.sync_copy(src_hbm.at[pl.ds(wid * gpt, gpt)], sidx)
        pltpu.sync_copy(dst_hbm.at[pl.ds(wid * gpt, gpt)], didx)
        plsc.subcore_barrier()

        def step(t, _):
            gath = [
                pltpu.async_copy(gsh.at[sidx.at[t * G_INNER + j]],
                                 rows.at[j], gsem)
                for j in range(G_INNER)
            ]
            for c in gath:
                c.wait()
            scat = [
                pltpu.async_copy(rows.at[j], acc.at[didx.at[t * G_INNER + j]],
                                 ssem, add=True)
                for j in range(G_INNER)
            ]
            for c in scat:
                c.wait()
            return 0
        lax.fori_loop(0, cpt, step, 0)
        plsc.subcore_barrier()

        pltpu.sync_copy(acc.at[pl.ds(row0, ROWS_PT)], zbuf)
        pltpu.sync_copy(zbuf, out_hbm.at[pl.ds(cid * NPAD + row0, ROWS_PT)])

    if layer == 1:
        return kern(src2d, dst2d, degp, h1)
    return kern(src2d, dst2d, degp, h1, agg1p, b1v)


def _dis_from_parts(degp):
    deg = degp[:NPAD].reshape(NPAD, 1) + degp[NPAD:].reshape(NPAD, 1) + 1.0
    return lax.rsqrt(deg)


def _tc_matmul1(x, w1p):
    def body(x_ref, w_ref, o_ref):
        o_ref[:N, :] = jnp.dot(x_ref[...], w_ref[...],
                               preferred_element_type=jnp.float32)
        o_ref[N:, :] = jnp.zeros((NPAD - N, LANES), jnp.float32)
    return pl.pallas_call(
        body,
        out_shape=jax.ShapeDtypeStruct((NPAD, LANES), jnp.float32),
    )(x, w1p)


def _tc_out(agg1p, agg2p, h1, degp, b1p, w2p, b2p):
    def body(a1_ref, a2_ref, h_ref, d_ref, b1_ref, w_ref, b2_ref, o_ref):
        dis = _dis_from_parts(d_ref[...])
        g1 = dis * h_ref[...]
        a1 = a1_ref[:NPAD, :] + a1_ref[NPAD:, :] + g1
        g2 = dis * jnp.maximum(dis * a1 + b1_ref[...], 0.0)
        agg = dis[:N] * (a2_ref[:N, :] + a2_ref[NPAD:NPAD + N, :] + g2[:N, :])
        o_ref[...] = jnp.dot(agg, w_ref[...],
                             preferred_element_type=jnp.float32) + b2_ref[...]
    return pl.pallas_call(
        body,
        out_shape=jax.ShapeDtypeStruct((N, D_OUT), jnp.float32),
    )(agg1p, agg2p, h1, degp, b1p, w2p, b2p)


def kernel(x, edge_index, W1, b1, W2, b2):
    e = edge_index.shape[1]
    epad = _pad_chunks(e)
    pad = epad - e

    src = jnp.concatenate(
        [edge_index[0].astype(jnp.int32), jnp.zeros((pad,), jnp.int32)])
    dst = jnp.concatenate(
        [edge_index[1].astype(jnp.int32),
         DUMMY + jnp.arange(pad, dtype=jnp.int32) % (NPAD - N)])
    src2d = src.reshape(-1, GROUP)
    dst2d = dst.reshape(-1, GROUP)

    w1p = jnp.zeros((D_IN, LANES), jnp.float32).at[:, :D_HID].set(W1)
    b1v = jnp.zeros((LANES,), jnp.float32).at[:D_HID].set(b1)
    w2p = jnp.zeros((LANES, D_OUT), jnp.float32).at[:D_HID].set(W2)
    b2p = b2.reshape(1, D_OUT)

    h1 = _tc_matmul1(x, w1p)
    degp = _sc_degree(dst2d)
    agg1 = _sc_aggregate(src2d, dst2d, degp, h1, None, None, layer=1)
    agg2 = _sc_aggregate(src2d, dst2d, degp, h1, agg1, b1v, layer=2)
    return _tc_out(agg1, agg2, h1, degp, b1v.reshape(1, LANES), w2p, b2p)


# speedup vs baseline: 1.7282x; 1.0790x over previous
import functools

import jax
import jax.numpy as jnp
from jax import lax
from jax.experimental import pallas as pl
from jax.experimental.pallas import tpu as pltpu
from jax.experimental.pallas import tpu_sc as plsc

N = 10000
D_IN = 128
D_HID = 15
D_OUT = 128

NC = 2
NS = 16
NW = NC * NS
LANES = 16

NPAD = 10240
ROWS_PT = NPAD // NS
DUMMY = N

GROUP = 128
G_INNER = 8
CHUNK = GROUP * G_INNER

_MESH = plsc.VectorSubcoreMesh(
    core_axis_name="c", subcore_axis_name="s", num_cores=NC, num_subcores=NS)
_SC_PARAMS = pltpu.CompilerParams(use_tc_tiling_on_sc=False,
                                  needs_layout_passes=False)


def _pad_chunks(e_total: int) -> int:
    return -(-e_total // (CHUNK * NW)) * CHUNK * NW


def _sc_degree(dst2d: jax.Array) -> jax.Array:
    gpt = dst2d.shape[0] // NW
    cpt = gpt // G_INNER

    @functools.partial(
        pl.kernel,
        out_type=jax.ShapeDtypeStruct((NC * NPAD,), jnp.float32),
        mesh=_MESH,
        compiler_params=_SC_PARAMS,
        scratch_types=dict(
            acc=pltpu.VMEM_SHARED((NPAD,), jnp.float32),
            didx=pltpu.VMEM((gpt, GROUP), jnp.int32),
            ones=pltpu.VMEM((GROUP,), jnp.float32),
            zbuf=pltpu.VMEM((ROWS_PT,), jnp.float32),
            ssem=pltpu.SemaphoreType.DMA,
        ),
    )
    def kern(dst_hbm, out_hbm, acc, didx, ones, zbuf, ssem):
        cid = lax.axis_index("c")
        sid = lax.axis_index("s")
        wid = cid * NS + sid

        def fill(i, _):
            zbuf[pl.ds(i * LANES, LANES)] = jnp.zeros((LANES,), jnp.float32)
            return 0
        lax.fori_loop(0, ROWS_PT // LANES, fill, 0)

        def fill1(i, _):
            ones[pl.ds(i * LANES, LANES)] = jnp.full((LANES,), 1.0, jnp.float32)
            return 0
        lax.fori_loop(0, GROUP // LANES, fill1, 0)

        pltpu.sync_copy(zbuf, acc.at[pl.ds(sid * ROWS_PT, ROWS_PT)])
        pltpu.sync_copy(dst_hbm.at[pl.ds(wid * gpt, gpt)], didx)
        plsc.subcore_barrier()

        def step(t, _):
            scat = [
                pltpu.async_copy(ones, acc.at[didx.at[t * G_INNER + j]], ssem,
                                 add=True)
                for j in range(G_INNER)
            ]
            for c in scat:
                c.wait()
            return 0
        lax.fori_loop(0, cpt, step, 0)
        plsc.subcore_barrier()

        pltpu.sync_copy(acc.at[pl.ds(sid * ROWS_PT, ROWS_PT)], zbuf)
        pltpu.sync_copy(zbuf, out_hbm.at[pl.ds(cid * NPAD + sid * ROWS_PT, ROWS_PT)])

    return kern(dst2d)


def _fisr(d):
    i = lax.bitcast_convert_type(d, jnp.int32)
    y = lax.bitcast_convert_type(jnp.int32(0x5F3759DF) - (i >> 1), jnp.float32)
    for _ in range(3):
        y = y * (1.5 - 0.5 * d * y * y)
    return y


def _sc_aggregate(src2d, dst2d, degp, h1, agg1p, b1v, layer: int):
    gpt = src2d.shape[0] // NW
    cpt = gpt // G_INNER

    scratch = dict(
        acc=pltpu.VMEM_SHARED((NPAD, LANES), jnp.float32),
        gsh=pltpu.VMEM_SHARED((NPAD, LANES), jnp.float32),
        sidx=pltpu.VMEM((gpt, GROUP), jnp.int32),
        didx=pltpu.VMEM((gpt, GROUP), jnp.int32),
        rows=pltpu.VMEM((G_INNER, GROUP, LANES), jnp.float32),
        zbuf=pltpu.VMEM((ROWS_PT, LANES), jnp.float32),
        hbuf=pltpu.VMEM((ROWS_PT, LANES), jnp.float32),
        d0v=pltpu.VMEM((ROWS_PT,), jnp.float32),
        d1v=pltpu.VMEM((ROWS_PT,), jnp.float32),
        disv=pltpu.VMEM((ROWS_PT,), jnp.float32),
        gsem=pltpu.SemaphoreType.DMA,
        ssem=pltpu.SemaphoreType.DMA,
    )
    if layer == 2:
        scratch.update(
            p0=pltpu.VMEM((ROWS_PT, LANES), jnp.float32),
            p1=pltpu.VMEM((ROWS_PT, LANES), jnp.float32),
            b1b=pltpu.VMEM((LANES,), jnp.float32),
        )

    @functools.partial(
        pl.kernel,
        out_type=jax.ShapeDtypeStruct((NC * NPAD, LANES), jnp.float32),
        mesh=_MESH,
        compiler_params=_SC_PARAMS,
        scratch_types=scratch,
    )
    def kern(*refs, acc, gsh, sidx, didx, rows, zbuf, hbuf, d0v, d1v, disv,
             gsem, ssem, p0=None, p1=None, b1b=None):
        if layer == 1:
            src_hbm, dst_hbm, degp_hbm, h1_hbm, out_hbm = refs
            a1_hbm = b1_hbm = None
        else:
            src_hbm, dst_hbm, degp_hbm, h1_hbm, a1_hbm, b1_hbm, out_hbm = refs
        cid = lax.axis_index("c")
        sid = lax.axis_index("s")
        wid = cid * NS + sid
        row0 = sid * ROWS_PT

        stage = [
            pltpu.async_copy(degp_hbm.at[pl.ds(row0, ROWS_PT)], d0v, gsem),
            pltpu.async_copy(degp_hbm.at[pl.ds(NPAD + row0, ROWS_PT)], d1v,
                             gsem),
            pltpu.async_copy(h1_hbm.at[pl.ds(row0, ROWS_PT)], hbuf, gsem),
        ]
        idxc = [
            pltpu.async_copy(src_hbm.at[pl.ds(wid * gpt, gpt)], sidx, ssem),
            pltpu.async_copy(dst_hbm.at[pl.ds(wid * gpt, gpt)], didx, ssem),
        ]
        if layer == 2:
            stage += [
                pltpu.async_copy(a1_hbm.at[pl.ds(row0, ROWS_PT)], p0, gsem),
                pltpu.async_copy(a1_hbm.at[pl.ds(NPAD + row0, ROWS_PT)], p1,
                                 gsem),
                pltpu.async_copy(b1_hbm, b1b, gsem),
            ]

        def fill(i, _):
            zbuf[i, :] = jnp.zeros((LANES,), jnp.float32)
            return 0
        lax.fori_loop(0, ROWS_PT, fill, 0)
        pltpu.sync_copy(zbuf, acc.at[pl.ds(row0, ROWS_PT)])
        for c in stage:
            c.wait()

        def dfill(i, _):
            d = d0v[pl.ds(i * LANES, LANES)] + d1v[pl.ds(i * LANES, LANES)] + 1.0
            disv[pl.ds(i * LANES, LANES)] = _fisr(d)
            return 0
        lax.fori_loop(0, ROWS_PT // LANES, dfill, 0)

        def srow(t, _):
            for u in range(8):
                r = t * 8 + u
                db = plsc.load_gather(disv, [jnp.full((LANES,), r, jnp.int32)])
                g1r = db * hbuf[r, :]
                if layer == 1:
                    zbuf[r, :] = g1r
                else:
                    v = (p0[r, :] + p1[r, :] + g1r) * db + b1b[:]
                    zbuf[r, :] = jnp.maximum(v, 0.0) * db
            return 0
        lax.fori_loop(0, ROWS_PT // 8, srow, 0)
        pltpu.sync_copy(zbuf, gsh.at[pl.ds(row0, ROWS_PT)])

        for c in idxc:
            c.wait()
        plsc.subcore_barrier()

        def step(t, _):
            gath = [
                pltpu.async_copy(gsh.at[sidx.at[t * G_INNER + j]],
                                 rows.at[j], gsem)
                for j in range(G_INNER)
            ]
            for c in gath:
                c.wait()
            scat = [
                pltpu.async_copy(rows.at[j], acc.at[didx.at[t * G_INNER + j]],
                                 ssem, add=True)
                for j in range(G_INNER)
            ]
            for c in scat:
                c.wait()
            return 0
        lax.fori_loop(0, cpt, step, 0)
        plsc.subcore_barrier()

        pltpu.sync_copy(acc.at[pl.ds(row0, ROWS_PT)], zbuf)
        pltpu.sync_copy(zbuf, out_hbm.at[pl.ds(cid * NPAD + row0, ROWS_PT)])

    if layer == 1:
        return kern(src2d, dst2d, degp, h1)
    return kern(src2d, dst2d, degp, h1, agg1p, b1v)


def _dis_from_parts(degp):
    deg = degp[:NPAD].reshape(NPAD, 1) + degp[NPAD:].reshape(NPAD, 1) + 1.0
    return lax.rsqrt(deg)


def _tc_matmul1(x, w1p):
    def body(x_ref, w_ref, o_ref):
        o_ref[:N, :] = jnp.dot(x_ref[...], w_ref[...],
                               preferred_element_type=jnp.float32)
        o_ref[N:, :] = jnp.zeros((NPAD - N, LANES), jnp.float32)
    return pl.pallas_call(
        body,
        out_shape=jax.ShapeDtypeStruct((NPAD, LANES), jnp.float32),
    )(x, w1p)


def _tc_out(agg1p, agg2p, h1, degp, b1p, w2p, b2p):
    def body(a1_ref, a2_ref, h_ref, d_ref, b1_ref, w_ref, b2_ref, o_ref):
        dis = _dis_from_parts(d_ref[...])
        g1 = dis * h_ref[...]
        a1 = a1_ref[:NPAD, :] + a1_ref[NPAD:, :] + g1
        g2 = dis * jnp.maximum(dis * a1 + b1_ref[...], 0.0)
        agg = dis[:N] * (a2_ref[:N, :] + a2_ref[NPAD:NPAD + N, :] + g2[:N, :])
        o_ref[...] = jnp.dot(agg, w_ref[...],
                             preferred_element_type=jnp.float32) + b2_ref[...]
    return pl.pallas_call(
        body,
        out_shape=jax.ShapeDtypeStruct((N, D_OUT), jnp.float32),
    )(agg1p, agg2p, h1, degp, b1p, w2p, b2p)


def kernel(x, edge_index, W1, b1, W2, b2):
    e = edge_index.shape[1]
    epad = _pad_chunks(e)
    pad = epad - e

    src = jnp.concatenate(
        [edge_index[0].astype(jnp.int32), jnp.zeros((pad,), jnp.int32)])
    dst = jnp.concatenate(
        [edge_index[1].astype(jnp.int32),
         DUMMY + jnp.arange(pad, dtype=jnp.int32) % (NPAD - N)])
    src2d = src.reshape(-1, GROUP)
    dst2d = dst.reshape(-1, GROUP)

    w1p = jnp.zeros((D_IN, LANES), jnp.float32).at[:, :D_HID].set(W1)
    b1v = jnp.zeros((LANES,), jnp.float32).at[:D_HID].set(b1)
    w2p = jnp.zeros((LANES, D_OUT), jnp.float32).at[:D_HID].set(W2)
    b2p = b2.reshape(1, D_OUT)

    h1 = _tc_matmul1(x, w1p)
    degp = _sc_degree(dst2d)
    agg1 = _sc_aggregate(src2d, dst2d, degp, h1, None, None, layer=1)
    agg2 = _sc_aggregate(src2d, dst2d, degp, h1, agg1, b1v, layer=2)
    return _tc_out(agg1, agg2, h1, degp, b1v.reshape(1, LANES), w2p, b2p)


# speedup vs baseline: 1.7371x; 1.0052x over previous
import functools

import jax
import jax.numpy as jnp
from jax import lax
from jax.experimental import pallas as pl
from jax.experimental.pallas import tpu as pltpu
from jax.experimental.pallas import tpu_sc as plsc

N = 10000
D_IN = 128
D_HID = 15
D_OUT = 128

NC = 2
NS = 16
NW = NC * NS
LANES = 16

NPAD = 10240
ROWS_PT = NPAD // NS
DUMMY = N

GROUP = 256
G_INNER = 4
CHUNK = GROUP * G_INNER

_MESH = plsc.VectorSubcoreMesh(
    core_axis_name="c", subcore_axis_name="s", num_cores=NC, num_subcores=NS)
_SC_PARAMS = pltpu.CompilerParams(use_tc_tiling_on_sc=False,
                                  needs_layout_passes=False)


def _pad_chunks(e_total: int) -> int:
    return -(-e_total // (CHUNK * NW)) * CHUNK * NW


def _sc_degree(dst2d: jax.Array) -> jax.Array:
    gpt = dst2d.shape[0] // NW
    cpt = gpt // G_INNER

    @functools.partial(
        pl.kernel,
        out_type=jax.ShapeDtypeStruct((NC * NPAD,), jnp.float32),
        mesh=_MESH,
        compiler_params=_SC_PARAMS,
        scratch_types=dict(
            acc=pltpu.VMEM_SHARED((NPAD,), jnp.float32),
            didx=pltpu.VMEM((gpt, GROUP), jnp.int32),
            ones=pltpu.VMEM((GROUP,), jnp.float32),
            zbuf=pltpu.VMEM((ROWS_PT,), jnp.float32),
            ssem=pltpu.SemaphoreType.DMA,
        ),
    )
    def kern(dst_hbm, out_hbm, acc, didx, ones, zbuf, ssem):
        cid = lax.axis_index("c")
        sid = lax.axis_index("s")
        wid = cid * NS + sid

        def fill(i, _):
            zbuf[pl.ds(i * LANES, LANES)] = jnp.zeros((LANES,), jnp.float32)
            return 0
        lax.fori_loop(0, ROWS_PT // LANES, fill, 0)

        def fill1(i, _):
            ones[pl.ds(i * LANES, LANES)] = jnp.full((LANES,), 1.0, jnp.float32)
            return 0
        lax.fori_loop(0, GROUP // LANES, fill1, 0)

        pltpu.sync_copy(zbuf, acc.at[pl.ds(sid * ROWS_PT, ROWS_PT)])
        pltpu.sync_copy(dst_hbm.at[pl.ds(wid * gpt, gpt)], didx)
        plsc.subcore_barrier()

        def step(t, _):
            scat = [
                pltpu.async_copy(ones, acc.at[didx.at[t * G_INNER + j]], ssem,
                                 add=True)
                for j in range(G_INNER)
            ]
            for c in scat:
                c.wait()
            return 0
        lax.fori_loop(0, cpt, step, 0)
        plsc.subcore_barrier()

        pltpu.sync_copy(acc.at[pl.ds(sid * ROWS_PT, ROWS_PT)], zbuf)
        pltpu.sync_copy(zbuf, out_hbm.at[pl.ds(cid * NPAD + sid * ROWS_PT, ROWS_PT)])

    return kern(dst2d)


def _fisr(d):
    i = lax.bitcast_convert_type(d, jnp.int32)
    y = lax.bitcast_convert_type(jnp.int32(0x5F3759DF) - (i >> 1), jnp.float32)
    for _ in range(3):
        y = y * (1.5 - 0.5 * d * y * y)
    return y


def _sc_aggregate(src2d, dst2d, degp, h1, agg1p, b1v, layer: int):
    gpt = src2d.shape[0] // NW
    cpt = gpt // G_INNER

    scratch = dict(
        acc=pltpu.VMEM_SHARED((NPAD, LANES), jnp.float32),
        gsh=pltpu.VMEM_SHARED((NPAD, LANES), jnp.float32),
        sidx=pltpu.VMEM((gpt, GROUP), jnp.int32),
        didx=pltpu.VMEM((gpt, GROUP), jnp.int32),
        rows=pltpu.VMEM((G_INNER, GROUP, LANES), jnp.float32),
        zbuf=pltpu.VMEM((ROWS_PT, LANES), jnp.float32),
        hbuf=pltpu.VMEM((ROWS_PT, LANES), jnp.float32),
        d0v=pltpu.VMEM((ROWS_PT,), jnp.float32),
        d1v=pltpu.VMEM((ROWS_PT,), jnp.float32),
        disv=pltpu.VMEM((ROWS_PT,), jnp.float32),
        gsem=pltpu.SemaphoreType.DMA,
        ssem=pltpu.SemaphoreType.DMA,
    )
    if layer == 2:
        scratch.update(
            p0=pltpu.VMEM((ROWS_PT, LANES), jnp.float32),
            p1=pltpu.VMEM((ROWS_PT, LANES), jnp.float32),
            b1b=pltpu.VMEM((LANES,), jnp.float32),
        )

    @functools.partial(
        pl.kernel,
        out_type=jax.ShapeDtypeStruct((NC * NPAD, LANES), jnp.float32),
        mesh=_MESH,
        compiler_params=_SC_PARAMS,
        scratch_types=scratch,
    )
    def kern(*refs, acc, gsh, sidx, didx, rows, zbuf, hbuf, d0v, d1v, disv,
             gsem, ssem, p0=None, p1=None, b1b=None):
        if layer == 1:
            src_hbm, dst_hbm, degp_hbm, h1_hbm, out_hbm = refs
            a1_hbm = b1_hbm = None
        else:
            src_hbm, dst_hbm, degp_hbm, h1_hbm, a1_hbm, b1_hbm, out_hbm = refs
        cid = lax.axis_index("c")
        sid = lax.axis_index("s")
        wid = cid * NS + sid
        row0 = sid * ROWS_PT

        stage = [
            pltpu.async_copy(degp_hbm.at[pl.ds(row0, ROWS_PT)], d0v, gsem),
            pltpu.async_copy(degp_hbm.at[pl.ds(NPAD + row0, ROWS_PT)], d1v,
                             gsem),
            pltpu.async_copy(h1_hbm.at[pl.ds(row0, ROWS_PT)], hbuf, gsem),
        ]
        idxc = [
            pltpu.async_copy(src_hbm.at[pl.ds(wid * gpt, gpt)], sidx, ssem),
            pltpu.async_copy(dst_hbm.at[pl.ds(wid * gpt, gpt)], didx, ssem),
        ]
        if layer == 2:
            stage += [
                pltpu.async_copy(a1_hbm.at[pl.ds(row0, ROWS_PT)], p0, gsem),
                pltpu.async_copy(a1_hbm.at[pl.ds(NPAD + row0, ROWS_PT)], p1,
                                 gsem),
                pltpu.async_copy(b1_hbm, b1b, gsem),
            ]

        def fill(i, _):
            zbuf[i, :] = jnp.zeros((LANES,), jnp.float32)
            return 0
        lax.fori_loop(0, ROWS_PT, fill, 0)
        pltpu.sync_copy(zbuf, acc.at[pl.ds(row0, ROWS_PT)])
        for c in stage:
            c.wait()

        def dfill(i, _):
            d = d0v[pl.ds(i * LANES, LANES)] + d1v[pl.ds(i * LANES, LANES)] + 1.0
            disv[pl.ds(i * LANES, LANES)] = _fisr(d)
            return 0
        lax.fori_loop(0, ROWS_PT // LANES, dfill, 0)

        def srow(t, _):
            for u in range(8):
                r = t * 8 + u
                db = plsc.load_gather(disv, [jnp.full((LANES,), r, jnp.int32)])
                g1r = db * hbuf[r, :]
                if layer == 1:
                    zbuf[r, :] = g1r
                else:
                    v = (p0[r, :] + p1[r, :] + g1r) * db + b1b[:]
                    zbuf[r, :] = jnp.maximum(v, 0.0) * db
            return 0
        lax.fori_loop(0, ROWS_PT // 8, srow, 0)
        pltpu.sync_copy(zbuf, gsh.at[pl.ds(row0, ROWS_PT)])

        for c in idxc:
            c.wait()
        plsc.subcore_barrier()

        def step(t, _):
            gath = [
                pltpu.async_copy(gsh.at[sidx.at[t * G_INNER + j]],
                                 rows.at[j], gsem)
                for j in range(G_INNER)
            ]
            for c in gath:
                c.wait()
            scat = [
                pltpu.async_copy(rows.at[j], acc.at[didx.at[t * G_INNER + j]],
                                 ssem, add=True)
                for j in range(G_INNER)
            ]
            for c in scat:
                c.wait()
            return 0
        lax.fori_loop(0, cpt, step, 0)
        plsc.subcore_barrier()

        pltpu.sync_copy(acc.at[pl.ds(row0, ROWS_PT)], zbuf)
        pltpu.sync_copy(zbuf, out_hbm.at[pl.ds(cid * NPAD + row0, ROWS_PT)])

    if layer == 1:
        return kern(src2d, dst2d, degp, h1)
    return kern(src2d, dst2d, degp, h1, agg1p, b1v)


def _dis_from_parts(degp):
    deg = degp[:NPAD].reshape(NPAD, 1) + degp[NPAD:].reshape(NPAD, 1) + 1.0
    return lax.rsqrt(deg)


def _tc_matmul1(x, w1p):
    def body(x_ref, w_ref, o_ref):
        o_ref[:N, :] = jnp.dot(x_ref[...], w_ref[...],
                               preferred_element_type=jnp.float32)
        o_ref[N:, :] = jnp.zeros((NPAD - N, LANES), jnp.float32)
    return pl.pallas_call(
        body,
        out_shape=jax.ShapeDtypeStruct((NPAD, LANES), jnp.float32),
    )(x, w1p)


def _tc_out(agg1p, agg2p, h1, degp, b1p, w2p, b2p):
    def body(a1_ref, a2_ref, h_ref, d_ref, b1_ref, w_ref, b2_ref, o_ref):
        dis = _dis_from_parts(d_ref[...])
        g1 = dis * h_ref[...]
        a1 = a1_ref[:NPAD, :] + a1_ref[NPAD:, :] + g1
        g2 = dis * jnp.maximum(dis * a1 + b1_ref[...], 0.0)
        agg = dis[:N] * (a2_ref[:N, :] + a2_ref[NPAD:NPAD + N, :] + g2[:N, :])
        o_ref[...] = jnp.dot(agg, w_ref[...],
                             preferred_element_type=jnp.float32) + b2_ref[...]
    return pl.pallas_call(
        body,
        out_shape=jax.ShapeDtypeStruct((N, D_OUT), jnp.float32),
    )(agg1p, agg2p, h1, degp, b1p, w2p, b2p)


def kernel(x, edge_index, W1, b1, W2, b2):
    e = edge_index.shape[1]
    epad = _pad_chunks(e)
    pad = epad - e

    src = jnp.concatenate(
        [edge_index[0].astype(jnp.int32), jnp.zeros((pad,), jnp.int32)])
    dst = jnp.concatenate(
        [edge_index[1].astype(jnp.int32),
         DUMMY + jnp.arange(pad, dtype=jnp.int32) % (NPAD - N)])
    src2d = src.reshape(-1, GROUP)
    dst2d = dst.reshape(-1, GROUP)

    w1p = jnp.zeros((D_IN, LANES), jnp.float32).at[:, :D_HID].set(W1)
    b1v = jnp.zeros((LANES,), jnp.float32).at[:D_HID].set(b1)
    w2p = jnp.zeros((LANES, D_OUT), jnp.float32).at[:D_HID].set(W2)
    b2p = b2.reshape(1, D_OUT)

    h1 = _tc_matmul1(x, w1p)
    degp = _sc_degree(dst2d)
    agg1 = _sc_aggregate(src2d, dst2d, degp, h1, None, None, layer=1)
    agg2 = _sc_aggregate(src2d, dst2d, degp, h1, agg1, b1v, layer=2)
    return _tc_out(agg1, agg2, h1, degp, b1v.reshape(1, LANES), w2p, b2p)


# speedup vs baseline: 1.8019x; 1.0373x over previous
import functools

import jax
import jax.numpy as jnp
from jax import lax
from jax.experimental import pallas as pl
from jax.experimental.pallas import tpu as pltpu
from jax.experimental.pallas import tpu_sc as plsc

N = 10000
D_IN = 128
D_HID = 15
D_OUT = 128

NC = 2
NS = 16
NW = NC * NS
LANES = 16

NPAD = 10240
ROWS_PT = NPAD // NS
DUMMY = N

GROUP = 256
G_INNER = 4
CHUNK = GROUP * G_INNER

_MESH = plsc.VectorSubcoreMesh(
    core_axis_name="c", subcore_axis_name="s", num_cores=NC, num_subcores=NS)
_SC_PARAMS = pltpu.CompilerParams(use_tc_tiling_on_sc=False,
                                  needs_layout_passes=False)


def _pad_chunks(e_total: int) -> int:
    return -(-e_total // (CHUNK * NW)) * CHUNK * NW


def _sc_degree(dst2d: jax.Array) -> jax.Array:
    gpt = dst2d.shape[0] // NW
    cpt = gpt // G_INNER

    @functools.partial(
        pl.kernel,
        out_type=jax.ShapeDtypeStruct((NC * NPAD,), jnp.float32),
        mesh=_MESH,
        compiler_params=_SC_PARAMS,
        scratch_types=dict(
            acc=pltpu.VMEM_SHARED((NPAD,), jnp.float32),
            didx=pltpu.VMEM((gpt, GROUP), jnp.int32),
            ones=pltpu.VMEM((GROUP,), jnp.float32),
            zbuf=pltpu.VMEM((ROWS_PT,), jnp.float32),
            ssem=pltpu.SemaphoreType.DMA,
        ),
    )
    def kern(dst_hbm, out_hbm, acc, didx, ones, zbuf, ssem):
        cid = lax.axis_index("c")
        sid = lax.axis_index("s")
        wid = cid * NS + sid

        def fill(i, _):
            zbuf[pl.ds(i * LANES, LANES)] = jnp.zeros((LANES,), jnp.float32)
            return 0
        lax.fori_loop(0, ROWS_PT // LANES, fill, 0)

        def fill1(i, _):
            ones[pl.ds(i * LANES, LANES)] = jnp.full((LANES,), 1.0, jnp.float32)
            return 0
        lax.fori_loop(0, GROUP // LANES, fill1, 0)

        pltpu.sync_copy(zbuf, acc.at[pl.ds(sid * ROWS_PT, ROWS_PT)])
        pltpu.sync_copy(dst_hbm.at[pl.ds(wid * gpt, gpt)], didx)
        plsc.subcore_barrier()

        scat = [
            pltpu.async_copy(ones, acc.at[didx.at[t * G_INNER + j]], ssem,
                             add=True)
            for t in range(cpt) for j in range(G_INNER)
        ]
        for c in scat:
            c.wait()
        plsc.subcore_barrier()

        pltpu.sync_copy(acc.at[pl.ds(sid * ROWS_PT, ROWS_PT)], zbuf)
        pltpu.sync_copy(zbuf, out_hbm.at[pl.ds(cid * NPAD + sid * ROWS_PT, ROWS_PT)])

    return kern(dst2d)


def _fisr(d):
    i = lax.bitcast_convert_type(d, jnp.int32)
    y = lax.bitcast_convert_type(jnp.int32(0x5F3759DF) - (i >> 1), jnp.float32)
    for _ in range(3):
        y = y * (1.5 - 0.5 * d * y * y)
    return y


def _sc_aggregate(src2d, dst2d, degp, h1, agg1p, b1v, layer: int):
    gpt = src2d.shape[0] // NW
    cpt = gpt // G_INNER

    scratch = dict(
        acc=pltpu.VMEM_SHARED((NPAD, LANES), jnp.float32),
        gsh=pltpu.VMEM_SHARED((NPAD, LANES), jnp.float32),
        sidx=pltpu.VMEM((gpt, GROUP), jnp.int32),
        didx=pltpu.VMEM((gpt, GROUP), jnp.int32),
        rows=pltpu.VMEM((2, G_INNER, GROUP, LANES), jnp.float32),
        zbuf=pltpu.VMEM((ROWS_PT, LANES), jnp.float32),
        hbuf=pltpu.VMEM((ROWS_PT, LANES), jnp.float32),
        d0v=pltpu.VMEM((ROWS_PT,), jnp.float32),
        d1v=pltpu.VMEM((ROWS_PT,), jnp.float32),
        disv=pltpu.VMEM((ROWS_PT,), jnp.float32),
        gsem=pltpu.SemaphoreType.DMA,
        gsem2=pltpu.SemaphoreType.DMA,
        ssem=pltpu.SemaphoreType.DMA,
        ssem2=pltpu.SemaphoreType.DMA,
    )
    if layer == 2:
        scratch.update(
            p0=pltpu.VMEM((ROWS_PT, LANES), jnp.float32),
            p1=pltpu.VMEM((ROWS_PT, LANES), jnp.float32),
            b1b=pltpu.VMEM((LANES,), jnp.float32),
        )

    @functools.partial(
        pl.kernel,
        out_type=jax.ShapeDtypeStruct((NC * NPAD, LANES), jnp.float32),
        mesh=_MESH,
        compiler_params=_SC_PARAMS,
        scratch_types=scratch,
    )
    def kern(*refs, acc, gsh, sidx, didx, rows, zbuf, hbuf, d0v, d1v, disv,
             gsem, gsem2, ssem, ssem2, p0=None, p1=None, b1b=None):
        if layer == 1:
            src_hbm, dst_hbm, degp_hbm, h1_hbm, out_hbm = refs
            a1_hbm = b1_hbm = None
        else:
            src_hbm, dst_hbm, degp_hbm, h1_hbm, a1_hbm, b1_hbm, out_hbm = refs
        cid = lax.axis_index("c")
        sid = lax.axis_index("s")
        wid = cid * NS + sid
        row0 = sid * ROWS_PT

        stage = [
            pltpu.async_copy(degp_hbm.at[pl.ds(row0, ROWS_PT)], d0v, gsem),
            pltpu.async_copy(degp_hbm.at[pl.ds(NPAD + row0, ROWS_PT)], d1v,
                             gsem),
            pltpu.async_copy(h1_hbm.at[pl.ds(row0, ROWS_PT)], hbuf, gsem),
        ]
        idxc = [
            pltpu.async_copy(src_hbm.at[pl.ds(wid * gpt, gpt)], sidx, ssem),
            pltpu.async_copy(dst_hbm.at[pl.ds(wid * gpt, gpt)], didx, ssem),
        ]
        if layer == 2:
            stage += [
                pltpu.async_copy(a1_hbm.at[pl.ds(row0, ROWS_PT)], p0, gsem),
                pltpu.async_copy(a1_hbm.at[pl.ds(NPAD + row0, ROWS_PT)], p1,
                                 gsem),
                pltpu.async_copy(b1_hbm, b1b, gsem),
            ]

        def fill(i, _):
            zbuf[i, :] = jnp.zeros((LANES,), jnp.float32)
            return 0
        lax.fori_loop(0, ROWS_PT, fill, 0)
        pltpu.sync_copy(zbuf, acc.at[pl.ds(row0, ROWS_PT)])
        for c in stage:
            c.wait()

        def dfill(i, _):
            d = d0v[pl.ds(i * LANES, LANES)] + d1v[pl.ds(i * LANES, LANES)] + 1.0
            disv[pl.ds(i * LANES, LANES)] = _fisr(d)
            return 0
        lax.fori_loop(0, ROWS_PT // LANES, dfill, 0)

        def srow(t, _):
            for u in range(8):
                r = t * 8 + u
                db = plsc.load_gather(disv, [jnp.full((LANES,), r, jnp.int32)])
                g1r = db * hbuf[r, :]
                if layer == 1:
                    zbuf[r, :] = g1r
                else:
                    v = (p0[r, :] + p1[r, :] + g1r) * db + b1b[:]
                    zbuf[r, :] = jnp.maximum(v, 0.0) * db
            return 0
        lax.fori_loop(0, ROWS_PT // 8, srow, 0)
        pltpu.sync_copy(zbuf, gsh.at[pl.ds(row0, ROWS_PT)])

        for c in idxc:
            c.wait()
        plsc.subcore_barrier()

        gsems = (gsem, gsem2)
        ssems = (ssem, ssem2)
        pend = [None, None]
        for t in range(cpt):
            b = t % 2
            if pend[b] is not None:
                for c in pend[b]:
                    c.wait()
            gath = [
                pltpu.async_copy(gsh.at[sidx.at[t * G_INNER + j]],
                                 rows.at[b, j], gsems[b])
                for j in range(G_INNER)
            ]
            for c in gath:
                c.wait()
            pend[b] = [
                pltpu.async_copy(rows.at[b, j],
                                 acc.at[didx.at[t * G_INNER + j]],
                                 ssems[b], add=True)
                for j in range(G_INNER)
            ]
        for lst in pend:
            if lst is not None:
                for c in lst:
                    c.wait()
        plsc.subcore_barrier()

        pltpu.sync_copy(acc.at[pl.ds(row0, ROWS_PT)], zbuf)
        pltpu.sync_copy(zbuf, out_hbm.at[pl.ds(cid * NPAD + row0, ROWS_PT)])

    if layer == 1:
        return kern(src2d, dst2d, degp, h1)
    return kern(src2d, dst2d, degp, h1, agg1p, b1v)


def _dis_from_parts(degp):
    deg = degp[:NPAD].reshape(NPAD, 1) + degp[NPAD:].reshape(NPAD, 1) + 1.0
    return lax.rsqrt(deg)


def _tc_matmul1(x, w1p):
    def body(x_ref, w_ref, o_ref):
        o_ref[:N, :] = jnp.dot(x_ref[...], w_ref[...],
                               preferred_element_type=jnp.float32)
        o_ref[N:, :] = jnp.zeros((NPAD - N, LANES), jnp.float32)
    return pl.pallas_call(
        body,
        out_shape=jax.ShapeDtypeStruct((NPAD, LANES), jnp.float32),
    )(x, w1p)


def _tc_out(agg1p, agg2p, h1, degp, b1p, w2p, b2p):
    def body(a1_ref, a2_ref, h_ref, d_ref, b1_ref, w_ref, b2_ref, o_ref):
        dis = _dis_from_parts(d_ref[...])
        g1 = dis * h_ref[...]
        a1 = a1_ref[:NPAD, :] + a1_ref[NPAD:, :] + g1
        g2 = dis * jnp.maximum(dis * a1 + b1_ref[...], 0.0)
        agg = dis[:N] * (a2_ref[:N, :] + a2_ref[NPAD:NPAD + N, :] + g2[:N, :])
        o_ref[...] = jnp.dot(agg, w_ref[...],
                             preferred_element_type=jnp.float32) + b2_ref[...]
    return pl.pallas_call(
        body,
        out_shape=jax.ShapeDtypeStruct((N, D_OUT), jnp.float32),
    )(agg1p, agg2p, h1, degp, b1p, w2p, b2p)


def kernel(x, edge_index, W1, b1, W2, b2):
    e = edge_index.shape[1]
    epad = _pad_chunks(e)
    pad = epad - e

    src = jnp.concatenate(
        [edge_index[0].astype(jnp.int32), jnp.zeros((pad,), jnp.int32)])
    dst = jnp.concatenate(
        [edge_index[1].astype(jnp.int32),
         DUMMY + jnp.arange(pad, dtype=jnp.int32) % (NPAD - N)])
    src2d = src.reshape(-1, GROUP)
    dst2d = dst.reshape(-1, GROUP)

    w1p = jnp.zeros((D_IN, LANES), jnp.float32).at[:, :D_HID].set(W1)
    b1v = jnp.zeros((LANES,), jnp.float32).at[:D_HID].set(b1)
    w2p = jnp.zeros((LANES, D_OUT), jnp.float32).at[:D_HID].set(W2)
    b2p = b2.reshape(1, D_OUT)

    h1 = _tc_matmul1(x, w1p)
    degp = _sc_degree(dst2d)
    agg1 = _sc_aggregate(src2d, dst2d, degp, h1, None, None, layer=1)
    agg2 = _sc_aggregate(src2d, dst2d, degp, h1, agg1, b1v, layer=2)
    return _tc_out(agg1, agg2, h1, degp, b1v.reshape(1, LANES), w2p, b2p)
